# Initial kernel scaffold; baseline (speedup 1.0000x reference)
import jax
import jax.numpy as jnp
from jax.experimental import pallas as pl


def kernel(x, mask, source, out):
    raise NotImplementedError("write your pallas kernel here")


# speedup vs baseline: 6.0511x; 6.0511x over previous
import functools

import jax
import jax.numpy as jnp
from jax import lax
from jax.experimental import pallas as pl
from jax.experimental.pallas import tpu as pltpu
from jax.experimental.pallas import tpu_sc as plsc

NC = 2
NS = 16
NW = NC * NS
L = 16


@functools.lru_cache(maxsize=None)
def _build(N: int):
    C = 16384
    assert N % (NW * C) == 0
    PW = N // NW
    K_CH = PW // C
    NCH = NW * K_CH
    assert K_CH == 32 and NCH % L == 0

    mesh = plsc.VectorSubcoreMesh(
        core_axis_name="c", subcore_axis_name="s",
        num_cores=NC, num_subcores=NS,
    )

    @functools.partial(
        pl.kernel,
        out_type=jax.ShapeDtypeStruct((NCH,), jnp.int32),
        mesh=mesh,
        scratch_types=[
            pltpu.VMEM((C,), jnp.int32),
            pltpu.VMEM((K_CH,), jnp.int32),
        ],
        compiler_params=pltpu.CompilerParams(needs_layout_passes=False),
    )
    def count_kernel(mask_hbm, counts_hbm, mvm, stage):
        wid = lax.axis_index("s") * NC + lax.axis_index("c")
        base = wid * PW
        iota = lax.iota(jnp.int32, L)
        zeros = jnp.zeros((L,), jnp.int32)

        def chunk_body(k, carry):
            lo, hi = carry
            pltpu.sync_copy(mask_hbm.at[pl.ds(base + k * C, C)], mvm)

            def vbody(i, acc):
                return acc + mvm[pl.ds(i * L, L)]

            acc = lax.fori_loop(0, C // L, vbody, zeros)
            tot = jnp.sum(acc)
            km = k % L
            lo = jnp.where((k < L) & (iota == km), tot, lo)
            hi = jnp.where((k >= L) & (iota == km), tot, hi)
            return lo, hi

        lo, hi = lax.fori_loop(0, K_CH, chunk_body, (zeros, zeros))
        stage[pl.ds(0, L)] = lo
        stage[pl.ds(L, L)] = hi
        pltpu.sync_copy(stage, counts_hbm.at[pl.ds(wid * K_CH, K_CH)])

    @functools.partial(
        pl.kernel,
        out_type=jax.ShapeDtypeStruct((N,), jnp.float32),
        mesh=mesh,
        scratch_types=[
            pltpu.VMEM((C,), jnp.int32),
            pltpu.VMEM((C,), jnp.float32),
            pltpu.VMEM((C + 8,), jnp.float32),
            pltpu.VMEM((NCH,), jnp.int32),
            pltpu.VMEM((NCH,), jnp.int32),
        ],
        compiler_params=pltpu.CompilerParams(needs_layout_passes=False),
    )
    def scatter_kernel(mask_hbm, x_hbm, src_hbm, counts_hbm, out_hbm,
                       mvm, xvm, svm, cvm, pvm):
        wid = lax.axis_index("s") * NC + lax.axis_index("c")
        base = wid * PW

        pltpu.sync_copy(counts_hbm, cvm)

        def pbody(i, carry):
            v = cvm[pl.ds(i * L, L)]
            cs = plsc.cumsum(v)
            pvm[pl.ds(i * L, L)] = carry + cs - v
            return carry + jnp.sum(v)

        lax.fori_loop(0, NCH // L, pbody, jnp.int32(0))

        iota = lax.iota(jnp.int32, L)

        def chunk_body(k, _):
            gcid = wid * K_CH + k
            start = base + k * C
            pvec = pvm[pl.ds((gcid // L) * L, L)]
            off = jnp.sum(jnp.where(iota == gcid % L, pvec, jnp.int32(0)))
            pltpu.sync_copy(mask_hbm.at[pl.ds(start, C)], mvm)
            pltpu.sync_copy(x_hbm.at[pl.ds(start, C)], xvm)
            wbase = jnp.maximum(
                jnp.minimum((off // 8) * 8, jnp.int32(N - (C + 8))),
                jnp.int32(0))
            wbase = pl.multiple_of(wbase, 8)
            pltpu.sync_copy(src_hbm.at[pl.ds(wbase, C + 8)], svm)
            delta = off - wbase

            def vbody(i, c0):
                mv = mvm[pl.ds(i * L, L)]
                xs = xvm[pl.ds(i * L, L)]
                cs = plsc.cumsum(mv)
                idx = jnp.maximum(delta + c0 + cs - 1, jnp.int32(0))
                mb = mv != 0
                g = plsc.load_gather(svm, [idx], mask=mb)
                xvm[pl.ds(i * L, L)] = jnp.where(mb, g, xs)
                return c0 + jnp.sum(mv)

            lax.fori_loop(0, C // L, vbody, jnp.int32(0))
            pltpu.sync_copy(xvm, out_hbm.at[pl.ds(start, C)])
            return _

        lax.fori_loop(0, K_CH, chunk_body, jnp.int32(0))

    return count_kernel, scatter_kernel


def kernel(x, mask, source, out):
    N = x.size
    mflat = mask.reshape(-1).astype(jnp.int32)
    xflat = x.reshape(-1)
    sflat = source.reshape(-1)
    count_kernel, scatter_kernel = _build(N)
    counts = count_kernel(mflat)
    res = scatter_kernel(mflat, xflat, sflat, counts)
    return res.reshape(x.shape)
